# baseline (speedup 1.0000x reference)
import functools

import jax
import jax.numpy as jnp
from jax.experimental import pallas as pl
from jax.experimental.pallas import tpu as pltpu

_BF = jnp.bfloat16


def _shift_zfill(a, d):
    L = a.shape[-1]
    z = jnp.zeros(a.shape[:-1] + (abs(d),), a.dtype)
    if d > 0:
        return jnp.concatenate([a[..., d:], z], axis=-1)
    return jnp.concatenate([z, a[..., :L + d]], axis=-1)


def _taps(ab, H, W):
    HW = H * W
    pos = jax.lax.broadcasted_iota(jnp.int32, (1, HW), 1)
    wpos = pos % W
    parts = []
    for dy in (-1, 0, 1):
        for dx in (-1, 0, 1):
            delta = dy * W + dx
            s = ab if delta == 0 else _shift_zfill(ab, delta)
            if dx == -1:
                s = jnp.where(wpos >= 1, s, jnp.zeros((), _BF))
            elif dx == 1:
                s = jnp.where(wpos <= W - 2, s, jnp.zeros((), _BF))
            parts.append(s)
    return parts


def _conv9(ab, w_ref, r0, C, H, W):
    parts = _taps(ab, H, W)
    acc = None
    for t0, t1 in ((0, 4), (4, 8), (8, 9)):
        seg = parts[t0] if t1 == t0 + 1 else jnp.concatenate(
            parts[t0:t1], axis=0)
        wseg = w_ref[r0:r0 + C, t0 * C:t1 * C]
        d = jnp.dot(wseg, seg, preferred_element_type=jnp.float32)
        acc = d if acc is None else acc + d
    return acc


def _fold(s, q, gamma, beta, count, eps):
    mean = s / count
    var = jnp.maximum(q / count - mean * mean, 0.0)
    scale = gamma * jax.lax.rsqrt(var + eps)
    shift = beta - mean * scale
    return scale, shift


def _mono_kernel(H, W, C, B, G, count, eps,
                 x_ref, w_ref, out_ref,
                 y1_scr, y2_scr, xb_scr, st1_scr, st2_scr, sc1_scr, sc2_scr):
    i = pl.program_id(0)

    @pl.when(i == 0)
    def _init():
        st1_scr[...] = jnp.zeros_like(st1_scr)
        st2_scr[...] = jnp.zeros_like(st2_scr)

    @pl.when(i < G)
    def _phase_a():
        s = jnp.zeros((C, 1), jnp.float32)
        q = jnp.zeros((C, 1), jnp.float32)
        for b in range(B):
            ab = x_ref[b].astype(_BF)

            @pl.when(i < G // 2)
            def _stash(b=b, ab=ab):
                xb_scr[i, b] = ab

            acc = _conv9(ab, w_ref, 0, C, H, W)
            y1_scr[i, b] = acc.astype(_BF)
            s = s + jnp.sum(acc, axis=1, keepdims=True)
            q = q + jnp.sum(acc * acc, axis=1, keepdims=True)
        st1_scr[:, 0:1] += s
        st1_scr[:, 1:2] += q

    @pl.when(i == G)
    def _fold1():
        pmt = jnp.transpose(
            w_ref[2 * C:2 * C + 8, 0:C].astype(jnp.float32))
        scale, shift = _fold(st1_scr[:, 0:1], st1_scr[:, 1:2],
                             pmt[:, 0:1], pmt[:, 1:2], count, eps)
        sc1_scr[:, 0:1] = scale.astype(_BF)
        sc1_scr[:, 1:2] = shift.astype(_BF)
        sc1_scr[:, 2:3] = pmt[:, 4:5].astype(_BF)

    @pl.when((i >= G) & (i < 2 * G))
    def _phase_b():
        j = i - G
        scale = sc1_scr[:, 0:1]
        shift = sc1_scr[:, 1:2]
        al = sc1_scr[:, 2:3]
        s = jnp.zeros((C, 1), jnp.float32)
        q = jnp.zeros((C, 1), jnp.float32)
        for b in range(B):
            z = y1_scr[j, b] * scale + shift
            ab = jnp.where(z >= 0, z, al * z)
            acc = _conv9(ab, w_ref, C, C, H, W)
            y2_scr[j, b] = acc.astype(_BF)
            s = s + jnp.sum(acc, axis=1, keepdims=True)
            q = q + jnp.sum(acc * acc, axis=1, keepdims=True)
        st2_scr[:, 0:1] += s
        st2_scr[:, 1:2] += q

    @pl.when(i == 2 * G)
    def _fold2():
        pmt = jnp.transpose(
            w_ref[2 * C:2 * C + 8, 0:C].astype(jnp.float32))
        scale, shift = _fold(st2_scr[:, 0:1], st2_scr[:, 1:2],
                             pmt[:, 2:3], pmt[:, 3:4], count, eps)
        sc2_scr[:, 0:1] = scale
        sc2_scr[:, 1:2] = shift

    @pl.when((i >= 2 * G) & (i < 2 * G + G // 2))
    def _phase_c_lo():
        k = i - 2 * G
        scale = sc2_scr[:, 0:1].reshape(1, C, 1)
        shift = sc2_scr[:, 1:2].reshape(1, C, 1)
        out_ref[...] = (xb_scr[k].astype(jnp.float32)
                        + y2_scr[k].astype(jnp.float32) * scale
                        + shift)

    @pl.when(i >= 2 * G + G // 2)
    def _phase_c_hi():
        k = i - 2 * G
        scale = sc2_scr[:, 0:1].reshape(1, C, 1)
        shift = sc2_scr[:, 1:2].reshape(1, C, 1)
        out_ref[...] = (x_ref[...]
                        + y2_scr[k].astype(jnp.float32) * scale
                        + shift)


def _prep_w(w_oihw, C):
    return jnp.transpose(w_oihw, (0, 2, 3, 1)).reshape(C, 9 * C)


def kernel(x, w1, b1, g1, be1, alpha, w2, b2, g2, be2, eps=1e-5):
    x = x.astype(jnp.float32)
    N, C, H, W = x.shape
    HW = H * W
    B = 4
    G = N // B
    count = float(N * HW)

    xg = x.reshape(G, B, C, HW)
    pm = jnp.stack([g1, be1, g2, be2,
                    jnp.broadcast_to(alpha, g1.shape),
                    jnp.zeros_like(g1), jnp.zeros_like(g1),
                    jnp.zeros_like(g1)]).astype(jnp.float32)
    w_all = jnp.concatenate([
        _prep_w(w1, C),
        _prep_w(w2, C),
        jnp.pad(pm, ((0, 0), (0, 8 * C))),
    ], axis=0).astype(_BF)

    x_map = lambda i: (jnp.where(i < G, i,
                                 jnp.where(i < 2 * G + G // 2, G - 1,
                                           i - 2 * G)), 0, 0, 0)
    out_map = lambda i: (jnp.where(i < 2 * G, 0, i - 2 * G), 0, 0, 0)

    mono = pl.pallas_call(
        functools.partial(_mono_kernel, H, W, C, B, G, count, eps),
        grid=(3 * G,),
        in_specs=[
            pl.BlockSpec((None, B, C, HW), x_map),
            pl.BlockSpec((2 * C + 8, 9 * C), lambda i: (0, 0)),
        ],
        out_specs=pl.BlockSpec((None, B, C, HW), out_map),
        out_shape=jax.ShapeDtypeStruct((G, B, C, HW), jnp.float32),
        scratch_shapes=[
            pltpu.VMEM((G, B, C, HW), _BF),
            pltpu.VMEM((G, B, C, HW), _BF),
            pltpu.VMEM((G // 2, B, C, HW), _BF),
            pltpu.VMEM((C, 2), jnp.float32),
            pltpu.VMEM((C, 2), jnp.float32),
            pltpu.VMEM((C, 4), _BF),
            pltpu.VMEM((C, 2), jnp.float32),
        ],
        compiler_params=pltpu.CompilerParams(
            dimension_semantics=("arbitrary",),
            vmem_limit_bytes=56 << 20,
        ),
    )

    out = mono(xg, w_all)
    return out.reshape(N, C, H, W)


# speedup vs baseline: 1.0184x; 1.0184x over previous
import functools

import jax
import jax.numpy as jnp
from jax.experimental import pallas as pl
from jax.experimental.pallas import tpu as pltpu

_BF = jnp.bfloat16


def _shift_zfill(a, d):
    L = a.shape[-1]
    z = jnp.zeros(a.shape[:-1] + (abs(d),), a.dtype)
    if d > 0:
        return jnp.concatenate([a[..., d:], z], axis=-1)
    return jnp.concatenate([z, a[..., :L + d]], axis=-1)


def _taps(ab, H, W):
    HW = H * W
    pos = jax.lax.broadcasted_iota(jnp.int32, (1, HW), 1)
    wpos = pos % W
    parts = []
    for dy in (-1, 0, 1):
        for dx in (-1, 0, 1):
            delta = dy * W + dx
            s = ab if delta == 0 else _shift_zfill(ab, delta)
            if dx == -1:
                s = jnp.where(wpos >= 1, s, jnp.zeros((), _BF))
            elif dx == 1:
                s = jnp.where(wpos <= W - 2, s, jnp.zeros((), _BF))
            parts.append(s)
    return parts


def _conv9(ab, w_ref, r0, C, H, W):
    parts = _taps(ab, H, W)
    acc = None
    for t0, t1 in ((0, 2), (2, 4), (4, 6), (6, 8), (8, 9)):
        seg = parts[t0] if t1 == t0 + 1 else jnp.concatenate(
            parts[t0:t1], axis=0)
        wseg = w_ref[r0:r0 + C, t0 * C:t1 * C]
        d = jnp.dot(wseg, seg, preferred_element_type=jnp.float32)
        acc = d if acc is None else acc + d
    return acc


def _fold(s, q, gamma, beta, count, eps):
    mean = s / count
    var = jnp.maximum(q / count - mean * mean, 0.0)
    scale = gamma * jax.lax.rsqrt(var + eps)
    shift = beta - mean * scale
    return scale, shift


def _mono_kernel(H, W, C, B, G, count, eps,
                 x_ref, w_ref, out_ref,
                 y1_scr, y2_scr, xb_scr, st1_scr, st2_scr, sc1_scr, sc2_scr):
    i = pl.program_id(0)

    @pl.when(i == 0)
    def _init():
        st1_scr[...] = jnp.zeros_like(st1_scr)
        st2_scr[...] = jnp.zeros_like(st2_scr)

    @pl.when(i < G)
    def _phase_a():
        s = jnp.zeros((C, 1), jnp.float32)
        q = jnp.zeros((C, 1), jnp.float32)
        for b in range(B):
            ab = x_ref[b].astype(_BF)

            @pl.when(i < G // 2)
            def _stash(b=b, ab=ab):
                xb_scr[i, b] = ab

            acc = _conv9(ab, w_ref, 0, C, H, W)
            y1_scr[i, b] = acc.astype(_BF)
            s = s + jnp.sum(acc, axis=1, keepdims=True)
            q = q + jnp.sum(acc * acc, axis=1, keepdims=True)
        st1_scr[:, 0:1] += s
        st1_scr[:, 1:2] += q

    @pl.when(i == G)
    def _fold1():
        pmt = jnp.transpose(
            w_ref[2 * C:2 * C + 8, 0:C].astype(jnp.float32))
        scale, shift = _fold(st1_scr[:, 0:1], st1_scr[:, 1:2],
                             pmt[:, 0:1], pmt[:, 1:2], count, eps)
        sc1_scr[:, 0:1] = scale.astype(_BF)
        sc1_scr[:, 1:2] = shift.astype(_BF)
        sc1_scr[:, 2:3] = pmt[:, 4:5].astype(_BF)

    @pl.when((i >= G) & (i < 2 * G))
    def _phase_b():
        j = i - G
        scale = sc1_scr[:, 0:1]
        shift = sc1_scr[:, 1:2]
        al = sc1_scr[:, 2:3]
        s = jnp.zeros((C, 1), jnp.float32)
        q = jnp.zeros((C, 1), jnp.float32)
        for b in range(B):
            z = y1_scr[j, b] * scale + shift
            ab = jnp.where(z >= 0, z, al * z)
            acc = _conv9(ab, w_ref, C, C, H, W)
            y2_scr[j, b] = acc.astype(_BF)
            s = s + jnp.sum(acc, axis=1, keepdims=True)
            q = q + jnp.sum(acc * acc, axis=1, keepdims=True)
        st2_scr[:, 0:1] += s
        st2_scr[:, 1:2] += q

    @pl.when(i == 2 * G)
    def _fold2():
        pmt = jnp.transpose(
            w_ref[2 * C:2 * C + 8, 0:C].astype(jnp.float32))
        scale, shift = _fold(st2_scr[:, 0:1], st2_scr[:, 1:2],
                             pmt[:, 2:3], pmt[:, 3:4], count, eps)
        sc2_scr[:, 0:1] = scale
        sc2_scr[:, 1:2] = shift

    @pl.when((i >= 2 * G) & (i < 2 * G + G // 2))
    def _phase_c_lo():
        k = i - 2 * G
        scale = sc2_scr[:, 0:1].reshape(1, C, 1)
        shift = sc2_scr[:, 1:2].reshape(1, C, 1)
        out_ref[...] = (xb_scr[k].astype(jnp.float32)
                        + y2_scr[k].astype(jnp.float32) * scale
                        + shift)

    @pl.when(i >= 2 * G + G // 2)
    def _phase_c_hi():
        k = i - 2 * G
        scale = sc2_scr[:, 0:1].reshape(1, C, 1)
        shift = sc2_scr[:, 1:2].reshape(1, C, 1)
        out_ref[...] = (x_ref[...]
                        + y2_scr[k].astype(jnp.float32) * scale
                        + shift)


def _prep_w(w_oihw, C):
    return jnp.transpose(w_oihw, (0, 2, 3, 1)).reshape(C, 9 * C)


def kernel(x, w1, b1, g1, be1, alpha, w2, b2, g2, be2, eps=1e-5):
    x = x.astype(jnp.float32)
    N, C, H, W = x.shape
    HW = H * W
    B = 4
    G = N // B
    count = float(N * HW)

    xg = x.reshape(G, B, C, HW)
    pm = jnp.stack([g1, be1, g2, be2,
                    jnp.broadcast_to(alpha, g1.shape),
                    jnp.zeros_like(g1), jnp.zeros_like(g1),
                    jnp.zeros_like(g1)]).astype(jnp.float32)
    w_all = jnp.concatenate([
        _prep_w(w1, C),
        _prep_w(w2, C),
        jnp.pad(pm, ((0, 0), (0, 8 * C))),
    ], axis=0).astype(_BF)

    x_map = lambda i: (jnp.where(i < G, i,
                                 jnp.where(i < 2 * G + G // 2, G - 1,
                                           i - 2 * G)), 0, 0, 0)
    out_map = lambda i: (jnp.where(i < 2 * G, 0, i - 2 * G), 0, 0, 0)

    mono = pl.pallas_call(
        functools.partial(_mono_kernel, H, W, C, B, G, count, eps),
        grid=(3 * G,),
        in_specs=[
            pl.BlockSpec((None, B, C, HW), x_map),
            pl.BlockSpec((2 * C + 8, 9 * C), lambda i: (0, 0)),
        ],
        out_specs=pl.BlockSpec((None, B, C, HW), out_map),
        out_shape=jax.ShapeDtypeStruct((G, B, C, HW), jnp.float32),
        scratch_shapes=[
            pltpu.VMEM((G, B, C, HW), _BF),
            pltpu.VMEM((G, B, C, HW), _BF),
            pltpu.VMEM((G // 2, B, C, HW), _BF),
            pltpu.VMEM((C, 2), jnp.float32),
            pltpu.VMEM((C, 2), jnp.float32),
            pltpu.VMEM((C, 4), _BF),
            pltpu.VMEM((C, 2), jnp.float32),
        ],
        compiler_params=pltpu.CompilerParams(
            dimension_semantics=("arbitrary",),
            vmem_limit_bytes=56 << 20,
        ),
    )

    out = mono(xg, w_all)
    return out.reshape(N, C, H, W)


# speedup vs baseline: 1.1042x; 1.0842x over previous
import functools

import jax
import jax.numpy as jnp
from jax.experimental import pallas as pl
from jax.experimental.pallas import tpu as pltpu

_BF = jnp.bfloat16


def _shift_zfill(a, d):
    L = a.shape[-1]
    z = jnp.zeros(a.shape[:-1] + (abs(d),), a.dtype)
    if d > 0:
        return jnp.concatenate([a[..., d:], z], axis=-1)
    return jnp.concatenate([z, a[..., :L + d]], axis=-1)


def _taps(ab, H, W):
    HW = H * W
    pos = jax.lax.broadcasted_iota(jnp.int32, (1, HW), 1)
    wpos = pos % W
    parts = []
    for dy in (-1, 0, 1):
        for dx in (-1, 0, 1):
            delta = dy * W + dx
            s = ab if delta == 0 else _shift_zfill(ab, delta)
            if dx == -1:
                s = jnp.where(wpos >= 1, s, jnp.zeros((), _BF))
            elif dx == 1:
                s = jnp.where(wpos <= W - 2, s, jnp.zeros((), _BF))
            parts.append(s)
    return parts


def _conv9(ab, w_ref, r0, C, H, W):
    parts = _taps(ab, H, W)
    acc = None
    for t0, t1 in ((0, 2), (2, 4), (4, 6), (6, 8), (8, 9)):
        seg = parts[t0] if t1 == t0 + 1 else jnp.concatenate(
            parts[t0:t1], axis=0)
        wseg = w_ref[r0:r0 + C, t0 * C:t1 * C]
        d = jnp.dot(wseg, seg, preferred_element_type=jnp.float32)
        acc = d if acc is None else acc + d
    return acc


def _fold(s, q, gamma, beta, count, eps):
    mean = s / count
    var = jnp.maximum(q / count - mean * mean, 0.0)
    scale = gamma * jax.lax.rsqrt(var + eps)
    shift = beta - mean * scale
    return scale, shift


def _mono_kernel(H, W, C, B, G, count, eps,
                 x_ref, w_ref, out_ref,
                 y1_scr, y2_scr, xb_scr, st1_scr, st2_scr, sc1_scr, sc2_scr):
    i = pl.program_id(0)

    @pl.when(i == 0)
    def _init():
        st1_scr[...] = jnp.zeros_like(st1_scr)
        st2_scr[...] = jnp.zeros_like(st2_scr)

    @pl.when(i < G)
    def _phase_a():
        s = jnp.zeros((C, 1), jnp.float32)
        q = jnp.zeros((C, 1), jnp.float32)
        for b in range(B):
            ab = x_ref[b].astype(_BF)
            xb_scr[i, b] = ab
            acc = _conv9(ab, w_ref, 0, C, H, W)
            y1_scr[i, b] = acc.astype(_BF)
            s = s + jnp.sum(acc, axis=1, keepdims=True)
            q = q + jnp.sum(acc * acc, axis=1, keepdims=True)
        st1_scr[:, 0:1] += s
        st1_scr[:, 1:2] += q

    @pl.when(i == G)
    def _fold1():
        pmt = jnp.transpose(
            w_ref[2 * C:2 * C + 8, 0:C].astype(jnp.float32))
        scale, shift = _fold(st1_scr[:, 0:1], st1_scr[:, 1:2],
                             pmt[:, 0:1], pmt[:, 1:2], count, eps)
        sc1_scr[:, 0:1] = scale.astype(_BF)
        sc1_scr[:, 1:2] = shift.astype(_BF)
        sc1_scr[:, 2:3] = pmt[:, 4:5].astype(_BF)

    @pl.when((i >= G) & (i < 2 * G))
    def _phase_b():
        j = i - G
        scale = sc1_scr[:, 0:1]
        shift = sc1_scr[:, 1:2]
        al = sc1_scr[:, 2:3]
        s = jnp.zeros((C, 1), jnp.float32)
        q = jnp.zeros((C, 1), jnp.float32)
        for b in range(B):
            z = y1_scr[j, b] * scale + shift
            ab = jnp.where(z >= 0, z, al * z)
            acc = _conv9(ab, w_ref, C, C, H, W)
            y2_scr[j, b] = acc.astype(_BF)
            s = s + jnp.sum(acc, axis=1, keepdims=True)
            q = q + jnp.sum(acc * acc, axis=1, keepdims=True)
        st2_scr[:, 0:1] += s
        st2_scr[:, 1:2] += q

    @pl.when(i == 2 * G)
    def _fold2():
        pmt = jnp.transpose(
            w_ref[2 * C:2 * C + 8, 0:C].astype(jnp.float32))
        scale, shift = _fold(st2_scr[:, 0:1], st2_scr[:, 1:2],
                             pmt[:, 2:3], pmt[:, 3:4], count, eps)
        sc2_scr[:, 0:1] = scale
        sc2_scr[:, 1:2] = shift

    @pl.when(i >= 2 * G)
    def _phase_c():
        k = i - 2 * G
        scale = sc2_scr[:, 0:1].reshape(1, C, 1)
        shift = sc2_scr[:, 1:2].reshape(1, C, 1)
        out_ref[...] = (xb_scr[k].astype(jnp.float32)
                        + y2_scr[k].astype(jnp.float32) * scale
                        + shift)


def _prep_w(w_oihw, C):
    return jnp.transpose(w_oihw, (0, 2, 3, 1)).reshape(C, 9 * C)


def kernel(x, w1, b1, g1, be1, alpha, w2, b2, g2, be2, eps=1e-5):
    x = x.astype(jnp.float32)
    N, C, H, W = x.shape
    HW = H * W
    B = 4
    G = N // B
    count = float(N * HW)

    xg = x.reshape(G, B, C, HW)
    pm = jnp.stack([g1, be1, g2, be2,
                    jnp.broadcast_to(alpha, g1.shape),
                    jnp.zeros_like(g1), jnp.zeros_like(g1),
                    jnp.zeros_like(g1)]).astype(jnp.float32)
    w_all = jnp.concatenate([
        _prep_w(w1, C),
        _prep_w(w2, C),
        jnp.pad(pm, ((0, 0), (0, 8 * C))),
    ], axis=0).astype(_BF)

    x_map = lambda i: (jnp.where(i < G, i, G - 1), 0, 0, 0)
    out_map = lambda i: (jnp.where(i < 2 * G, 0, i - 2 * G), 0, 0, 0)

    mono = pl.pallas_call(
        functools.partial(_mono_kernel, H, W, C, B, G, count, eps),
        grid=(3 * G,),
        in_specs=[
            pl.BlockSpec((None, B, C, HW), x_map),
            pl.BlockSpec((2 * C + 8, 9 * C), lambda i: (0, 0)),
        ],
        out_specs=pl.BlockSpec((None, B, C, HW), out_map),
        out_shape=jax.ShapeDtypeStruct((G, B, C, HW), jnp.float32),
        scratch_shapes=[
            pltpu.VMEM((G, B, C, HW), _BF),
            pltpu.VMEM((G, B, C, HW), _BF),
            pltpu.VMEM((G, B, C, HW), _BF),
            pltpu.VMEM((C, 2), jnp.float32),
            pltpu.VMEM((C, 2), jnp.float32),
            pltpu.VMEM((C, 4), _BF),
            pltpu.VMEM((C, 2), jnp.float32),
        ],
        compiler_params=pltpu.CompilerParams(
            dimension_semantics=("arbitrary",),
            vmem_limit_bytes=(58 << 20) + (1 << 19),
        ),
    )

    out = mono(xg, w_all)
    return out.reshape(N, C, H, W)
